# baseline (speedup 1.0000x reference)
import jax
import jax.numpy as jnp
from jax.experimental import pallas as pl
from jax.experimental.pallas import tpu as pltpu

NUM_PATCHES = 1024
EMBED_DIM = 768
BATCH = 64

BB = 8
EE = 2


def _body(pos_ref, patch_ref, out_ref, tpos_ref):
    @pl.when((pl.program_id(0) == 0) & (pl.program_id(1) == 0))
    def _init():
        tpos_ref[...] = pos_ref[...].T

    e = pl.program_id(1)
    out_ref[...] = patch_ref[...] + tpos_ref[pl.ds(e * (EMBED_DIM // EE), EMBED_DIM // EE), :][None, :, :]


def kernel(patch, pos_table):
    return pl.pallas_call(
        _body,
        grid=(BATCH // BB, EE),
        in_specs=[
            pl.BlockSpec((NUM_PATCHES, EMBED_DIM), lambda b, e: (0, 0)),
            pl.BlockSpec((BB, EMBED_DIM // EE, NUM_PATCHES), lambda b, e: (b, e, 0)),
        ],
        out_specs=pl.BlockSpec((BB, EMBED_DIM // EE, NUM_PATCHES), lambda b, e: (b, e, 0)),
        out_shape=jax.ShapeDtypeStruct((BATCH, EMBED_DIM, NUM_PATCHES), patch.dtype),
        scratch_shapes=[pltpu.VMEM((EMBED_DIM, NUM_PATCHES), jnp.float32)],
        compiler_params=pltpu.CompilerParams(
            dimension_semantics=("arbitrary", "arbitrary"),
        ),
    )(pos_table, patch)
